# baseline (speedup 1.0000x reference)
import functools

import jax
import jax.numpy as jnp
from jax import lax
from jax.experimental import pallas as pl
from jax.experimental.pallas import tpu as pltpu
from jax.experimental.pallas import tpu_sc as plsc

N = 10000
E = 320000
B = 64
W = 128

_NT = 16
_EPT = E // _NT
_K = 80
_NCHUNK = _EPT // _K
_ROWS_MAIN = 624
_ROWS_TAIL = N - 16 * _ROWS_MAIN
_ZROWS = 48


def _sc_msg_pass_fn(n_rounds):
    mesh = plsc.VectorSubcoreMesh(core_axis_name="c", subcore_axis_name="s")

    RI = 6
    RD = 4
    MOD = 12

    @functools.partial(
        pl.kernel,
        out_type=tuple(jax.ShapeDtypeStruct((N, W), jnp.float32)
                       for _ in range(2 * n_rounds)),
        mesh=mesh,
        scratch_types=(
            [pltpu.VMEM((_K,), jnp.int32)] * (2 * RI)
            + [pltpu.VMEM((_K, W), jnp.float32)] * RD
            + [pltpu.VMEM((_ZROWS, W), jnp.float32),
               pltpu.VMEM_SHARED((N, W), jnp.float32)]
            + [pltpu.SemaphoreType.DMA] * (2 * RI + 2 * RD)
        ),
    )
    def msg_pass(*args):
        hs_c = args[0:n_rounds]
        hs_s = args[n_rounds:2 * n_rounds]
        src_c, dst_c, src_s, dst_s = args[2 * n_rounds:2 * n_rounds + 4]
        outs_c = args[2 * n_rounds + 4:3 * n_rounds + 4]
        outs_s = args[3 * n_rounds + 4:4 * n_rounds + 4]
        scr = args[4 * n_rounds + 4:]

        sis = scr[0:RI]
        dis = scr[RI:2 * RI]
        rowss = scr[2 * RI:2 * RI + RD]
        zbuf = scr[2 * RI + RD]
        acc = scr[2 * RI + RD + 1]
        sems = scr[2 * RI + RD + 2:]
        sss = sems[0:RI]
        dss = sems[RI:2 * RI]
        gss = sems[2 * RI:2 * RI + RD]
        css = sems[2 * RI + RD:]

        c = lax.axis_index("c")
        s = lax.axis_index("s")
        row0 = s * _ROWS_MAIN
        base0 = s * _EPT

        def load_idx(src, dst, j, ri):
            off = base0 + j * _K
            pltpu.async_copy(src.at[pl.ds(off, _K)], sis[ri], sss[ri])
            pltpu.async_copy(dst.at[pl.ds(off, _K)], dis[ri], dss[ri])

        def start_gather(h, src, dst, j, ri, rd):
            off = base0 + j * _K
            pltpu.make_async_copy(src.at[pl.ds(off, _K)], sis[ri],
                                  sss[ri]).wait()
            pltpu.make_async_copy(dst.at[pl.ds(off, _K)], dis[ri],
                                  dss[ri]).wait()
            pltpu.async_copy(h.at[sis[ri]], rowss[rd], gss[rd])

        def finish(h, j, r):
            ri, rd = r % RI, r % RD
            pltpu.make_async_copy(h.at[sis[ri]], rowss[rd],
                                  gss[rd]).wait()

            @pl.when(j >= 1)
            def _():
                pri, prd = (r - 1) % RI, (r - 1) % RD
                pltpu.make_async_copy(rowss[prd], acc.at[dis[pri]],
                                      css[prd]).wait()
            pltpu.async_copy(rowss[rd], acc.at[dis[ri]], css[rd], add=True)

        def prologue(h, src, dst):
            for jj in range(3):
                load_idx(src, dst, jj, jj)
            start_gather(h, src, dst, 0, 0, 0)
            start_gather(h, src, dst, 1, 1, 1)

        def edge_loop(h, src, dst):
            def step(j, r):
                @pl.when(j + 3 < _NCHUNK)
                def _():
                    load_idx(src, dst, j + 3, (r + 3) % RI)

                @pl.when(j + 2 < _NCHUNK)
                def _():
                    start_gather(h, src, dst, j + 2, (r + 2) % RI,
                                 (r + 2) % RD)
                finish(h, j, r)

            def chunk(j, _):
                for r in range(MOD):
                    @pl.when(j % MOD == r)
                    def _(r=r):
                        step(j, r)
                return 0
            lax.fori_loop(0, _NCHUNK, chunk, 0)

            jj = _NCHUNK - 1
            pltpu.make_async_copy(rowss[jj % RD], acc.at[dis[jj % RI]],
                                  css[jj % RD]).wait()

        def zero_acc():
            def zero_blk(i, _):
                pltpu.sync_copy(zbuf,
                                acc.at[pl.ds(row0 + _ZROWS * i, _ZROWS)])
                return 0
            lax.fori_loop(0, _ROWS_MAIN // _ZROWS, zero_blk, 0)

            @pl.when(s == _NT - 1)
            def _():
                pltpu.sync_copy(zbuf.at[pl.ds(0, _ROWS_TAIL)],
                                acc.at[pl.ds(16 * _ROWS_MAIN, _ROWS_TAIL)])

        def copy_out(o):
            pltpu.sync_copy(acc.at[pl.ds(row0, _ROWS_MAIN)],
                            o.at[pl.ds(row0, _ROWS_MAIN)])

            @pl.when(s == _NT - 1)
            def _():
                pltpu.sync_copy(acc.at[pl.ds(16 * _ROWS_MAIN, _ROWS_TAIL)],
                                o.at[pl.ds(16 * _ROWS_MAIN, _ROWS_TAIL)])

        def run_rounds(hs, src, dst, outs):
            for r in range(n_rounds):
                if r == 0:
                    prologue(hs[0], src, dst)
                    zero_acc()
                else:
                    copy_out(outs[r - 1])
                    zero_acc()
                    prologue(hs[r], src, dst)
                plsc.subcore_barrier()
                edge_loop(hs[r], src, dst)
                plsc.subcore_barrier()
            copy_out(outs[-1])

        zero16 = jnp.zeros((16,), jnp.float32)

        def zb(i, _):
            r = i // (W // 16)
            col = (i % (W // 16)) * 16
            zbuf[r, pl.ds(col, 16)] = zero16
            return 0
        lax.fori_loop(0, _ZROWS * W // 16, zb, 0)

        @pl.when(c == 0)
        def _():
            run_rounds(hs_c, src_c, dst_c, outs_c)

        @pl.when(c == 1)
        def _():
            run_rounds(hs_s, src_s, dst_s, outs_s)

    return msg_pass


_SC_CACHE = {}


def _msg_pass(n_rounds):
    if n_rounds not in _SC_CACHE:
        _SC_CACHE[n_rounds] = _sc_msg_pass_fn(n_rounds)
    return _SC_CACHE[n_rounds]


def _pre_kernel(xc_ref, xs_ref, cw00_ref, cw10_ref, sw00_ref, sw10_ref,
                hc0_ref, hc1_ref, hs0_ref, hs1_ref):
    xc = xc_ref[...]
    xs = xs_ref[...]
    hc0_ref[...] = jnp.dot(xc, cw00_ref[...])
    hc1_ref[...] = jnp.dot(xc, cw10_ref[...])
    hs0_ref[...] = jnp.dot(xs, sw00_ref[...])
    hs1_ref[...] = jnp.dot(xs, sw10_ref[...])


def _mid_kernel(mc0_ref, mc1_ref, ms0_ref, ms1_ref,
                cw01_ref, cw11_ref, sw01_ref, sw11_ref, gc_ref, gs_ref):
    def one(m0_ref, m1_ref, w01_ref, w11_ref, g_ref):
        c0 = jnp.maximum(m0_ref[...], 0.0)
        c1 = jnp.maximum(m1_ref[...], 0.0)
        g_ref[...] = jnp.concatenate(
            [jnp.dot(c0, w01_ref[...]), jnp.dot(c1, w11_ref[...])], axis=1)

    one(mc0_ref, mc1_ref, cw01_ref, cw11_ref, gc_ref)
    one(ms0_ref, ms1_ref, sw01_ref, sw11_ref, gs_ref)


def _head_kernel(pc_ref, ps_ref, cb_ref, sb_ref,
                 cw0_ref, cb0_ref, cw1_ref, cb1_ref,
                 sw0_ref, sb0_ref, sw1_ref, sb1_ref,
                 f1w_ref, f1b_ref, f2w_ref, f2b_ref, out_ref):
    iota_b = lax.broadcasted_iota(jnp.int32, (B, N), 0)
    pc = (cb_ref[...] == iota_b).astype(jnp.float32)
    ps = (sb_ref[...] == iota_b).astype(jnp.float32)

    m2c = jnp.maximum(pc_ref[...], 0.0)
    m2s = jnp.maximum(ps_ref[...], 0.0)

    hp = jax.lax.Precision.HIGHEST
    repc = jnp.dot(pc, m2c, precision=hp)
    reps = jnp.dot(ps, m2s, precision=hp)

    def fc(r, w_ref, b_ref):
        return jnp.maximum(jnp.dot(r, w_ref[...]) + b_ref[...], 0.0)

    ind = jnp.concatenate([
        fc(repc[:, :64], cw0_ref, cb0_ref),
        fc(repc[:, 64:], cw1_ref, cb1_ref),
        fc(reps[:, :64], sw0_ref, sb0_ref),
        fc(reps[:, 64:], sw1_ref, sb1_ref),
    ], axis=1)
    hg = jnp.maximum(jnp.dot(ind, f1w_ref[...]) + f1b_ref[...], 0.0)
    out_ref[...] = jnp.dot(hg, f2w_ref[...]) + f2b_ref[...]


def kernel(chr_x, chr_edge_index, chr_x_batch, slv_x, slv_edge_index,
           slv_x_batch, pseudo_batch,
           chr_W00, chr_W01, chr_W10, chr_W11,
           slv_W00, slv_W01, slv_W10, slv_W11,
           cfc_w0, cfc_b0, cfc_w1, cfc_b1,
           sfc_w0, sfc_b0, sfc_w1, sfc_b1,
           fc1_w, fc1_b, fc2_w, fc2_b):
    del pseudo_batch
    csrc, cdst = chr_edge_index[0], chr_edge_index[1]
    ssrc, sdst = slv_edge_index[0], slv_edge_index[1]

    hc0, hc1, hs0, hs1 = pl.pallas_call(
        _pre_kernel,
        out_shape=tuple(jax.ShapeDtypeStruct((N, W), jnp.float32)
                        for _ in range(4)),
    )(chr_x, slv_x, chr_W00, chr_W10, slv_W00, slv_W10)

    mc0, mc1, ms0, ms1 = _msg_pass(2)(hc0, hc1, hs0, hs1,
                                      csrc, cdst, ssrc, sdst)

    g_chr, g_slv = pl.pallas_call(
        _mid_kernel,
        out_shape=(jax.ShapeDtypeStruct((N, W), jnp.float32),
                   jax.ShapeDtypeStruct((N, W), jnp.float32)),
    )(mc0, mc1, ms0, ms1, chr_W01, chr_W11, slv_W01, slv_W11)

    p_chr, p_slv = _msg_pass(1)(g_chr, g_slv, csrc, cdst, ssrc, sdst)

    out = pl.pallas_call(
        _head_kernel,
        out_shape=jax.ShapeDtypeStruct((B, 1), jnp.float32),
    )(p_chr, p_slv,
      chr_x_batch.reshape(1, N), slv_x_batch.reshape(1, N),
      cfc_w0, cfc_b0.reshape(1, -1), cfc_w1, cfc_b1.reshape(1, -1),
      sfc_w0, sfc_b0.reshape(1, -1), sfc_w1, sfc_b1.reshape(1, -1),
      fc1_w, fc1_b.reshape(1, -1), fc2_w, fc2_b.reshape(1, 1))
    return out
